# baseline (speedup 1.0000x reference)
import functools

import jax
import jax.numpy as jnp
from jax import lax
from jax.experimental import pallas as pl
from jax.experimental.pallas import tpu as pltpu
from jax.experimental.pallas import tpu_sc as plsc

_D = 64
_CH = 128

_NC = 2
_NS = 16
_NW = _NC * _NS


def _make_gather(n_fields: int, batch: int, n_pair_rows: int):
    n_units = n_fields * (batch // _CH)
    assert n_units % _NW == 0
    upw = n_units // _NW
    bchunks = batch // _CH

    mesh = plsc.VectorSubcoreMesh(core_axis_name="c", subcore_axis_name="s")

    scratch = [
        pltpu.VMEM((_CH,), jnp.int32),
        pltpu.VMEM((_CH,), jnp.int32),
        pltpu.VMEM((_CH,), jnp.int32),
        pltpu.VMEM((_CH, 2 * _D), jnp.float32),
        pltpu.VMEM((_D, _CH), jnp.float32),
        pltpu.SemaphoreType.DMA,
        pltpu.SemaphoreType.DMA,
    ]

    @functools.partial(
        pl.kernel,
        out_type=jax.ShapeDtypeStruct((n_fields, _D, batch), jnp.float32),
        mesh=mesh,
        scratch_types=scratch,
        compiler_params=pltpu.CompilerParams(use_tc_tiling_on_sc=True,
                                             needs_layout_passes=False),
    )
    def gather_kernel(ids_hbm, table_hbm, out_hbm,
                      ids_v, idx_v, col_v, pair_v, outb_v, gsem, osem):
        wid = lax.axis_index("s") * _NC + lax.axis_index("c")
        u0 = wid * upw
        iota16 = lax.iota(jnp.int32, 16)

        def body(i, carry):
            u = u0 + i
            f = u // bchunks
            b0 = (u % bchunks) * _CH

            pltpu.sync_copy(ids_hbm.at[f, pl.ds(b0, _CH)], ids_v)
            for g in range(_CH // 16):
                sl = pl.ds(g * 16, 16)
                idv = ids_v[sl]
                idx_v[sl] = lax.shift_right_logical(idv, 1)
                col_v[sl] = (idv & 1) * _D

            pltpu.async_copy(table_hbm.at[idx_v], pair_v, gsem).wait()

            for g in range(_CH // 16):
                sl = pl.ds(g * 16, 16)
                rowv = iota16 + (g * 16)
                colv = col_v[sl]
                for d in range(_D):
                    outb_v[d, sl] = plsc.load_gather(pair_v, [rowv, colv + d])

            pltpu.async_copy(
                outb_v, out_hbm.at[f, :, pl.ds(b0, _CH)], osem).wait()
            return carry

        lax.fori_loop(0, upw, body, 0, unroll=False)

    return gather_kernel


def kernel(ids, table):
    batch, n_fields = ids.shape
    n_rows, d = table.shape
    ids_t = ids.T.astype(jnp.int32)
    table2 = table.reshape(n_rows // 2, 2 * d)
    out_t = _make_gather(n_fields, batch, n_rows // 2)(ids_t, table2)
    return out_t.transpose(2, 0, 1)
